# baseline (speedup 1.0000x reference)
import functools

import jax
import jax.numpy as jnp
from jax import lax
from jax.experimental import pallas as pl
from jax.experimental.pallas import tpu as pltpu
from jax.experimental.pallas import tpu_sc as plsc

B = 16384
D = 16
NROWS = 1000000
_NC = 2
_NS = 16
_NW = _NC * _NS
_BPW = B // _NW
_CHUNK = 64
_NCHUNK = _BPW // _CHUNK
_KG = _CHUNK // 16

_NP = D // 2
_NB = (NROWS + 1023) // 1024
_FPAD = _NB * 1024
_FROWS = _FPAD * _NP // 16
_JC = [j * (_FPAD // 16) for j in range(_NP)]

_mesh = plsc.VectorSubcoreMesh(core_axis_name="c", subcore_axis_name="s")


def _burst_base(iv):
    return iv >> 4


def _make_gather(mode):
    three = mode == "three"
    n_out = 2 if three else 1
    n_tab = 3 if three else 1
    n_idx = 2 if three else 1

    @functools.partial(
        pl.kernel,
        mesh=_mesh,
        compiler_params=pltpu.CompilerParams(
            use_tc_tiling_on_sc=False, needs_layout_passes=False),
        out_type=[jax.ShapeDtypeStruct((B // 8, 128), jnp.float32)] * n_out,
        scratch_types=(
            [pltpu.VMEM((_BPW,), jnp.int32)] * n_idx
            + [pltpu.VMEM((2, _CHUNK * _NP), jnp.int32)] * n_idx
            + [pltpu.VMEM((2, _CHUNK * _NP, 16), jnp.float32)] * n_tab
            + [pltpu.VMEM((_BPW // 8, 128), jnp.float32)] * n_out
            + [pltpu.SemaphoreType.DMA]
        ),
    )
    def gather(*args):
        pos = 0

        def take(n):
            nonlocal pos
            out = args[pos:pos + n]
            pos += n
            return out

        idx_hbm = take(n_idx)
        tabs = take(n_tab)
        outs = take(n_out)
        idx_v = take(n_idx)
        ibs = take(n_idx)
        gs = take(n_tab)
        vs = take(n_out)
        (sem,) = take(1)

        wid = lax.axis_index("s") * _NC + lax.axis_index("c")
        base = wid * _BPW
        for h, v in zip(idx_hbm, idx_v):
            pltpu.sync_copy(h.at[pl.ds(base, _BPW)], v)

        iota = lax.iota(jnp.int32, 16)
        tab_idx = (0, 1, 0) if three else (0,)

        def unpack(w):
            wi = plsc.bitcast(w, jnp.int32)
            lo = plsc.bitcast(wi << 16, jnp.float32)
            hi = plsc.bitcast(wi & jnp.int32(-65536), jnp.float32)
            return lo, hi

        def build_and_fire(c):
            p = c % 2
            c0 = c * _CHUNK
            for k in range(_KG):
                for ib, iv_ref in zip(ibs, idx_v):
                    bv = _burst_base(iv_ref[pl.ds(c0 + k * 16, 16)])
                    for j in range(_NP):
                        ib[p, pl.ds(j * _CHUNK + k * 16, 16)] = bv + _JC[j]
            return [pltpu.async_copy(t.at[ibs[s].at[p]], g.at[p], sem)
                    for t, g, s in zip(tabs, gs, tab_idx)]

        def extract(c):
            p = c % 2
            c0 = c * _CHUNK
            for k in range(_KG):
                lanes = [v[pl.ds(c0 + k * 16, 16)] & 15 for v in idx_v]
                rows = c0 + k * 16 + iota
                r2 = rows >> 3
                cb = (rows & 7) << 4
                for j in range(_NP):
                    gr = j * _CHUNK + k * 16 + iota
                    if three:
                        u0, u1 = unpack(
                            plsc.load_gather(gs[0].at[p], [gr, lanes[0]]))
                        i0, i1 = unpack(
                            plsc.load_gather(gs[1].at[p], [gr, lanes[1]]))
                        m0, m1 = unpack(
                            plsc.load_gather(gs[2].at[p], [gr, lanes[0]]))
                        plsc.store_scatter(vs[0], [r2, cb + j], u0 * i0)
                        plsc.store_scatter(vs[0], [r2, cb + (j + _NP)],
                                           u1 * i1)
                        plsc.store_scatter(vs[1], [r2, cb + j], m0)
                        plsc.store_scatter(vs[1], [r2, cb + (j + _NP)], m1)
                    else:
                        x0, x1 = unpack(
                            plsc.load_gather(gs[0].at[p], [gr, lanes[0]]))
                        plsc.store_scatter(vs[0], [r2, cb + j], x0)
                        plsc.store_scatter(vs[0], [r2, cb + (j + _NP)], x1)

        cps = build_and_fire(0)
        for c in range(1, _NCHUNK):
            nxt = build_and_fire(c)
            for cp in cps:
                cp.wait()
            extract(c - 1)
            cps = nxt
        for cp in cps:
            cp.wait()
        extract(_NCHUNK - 1)

        out_sl = pl.ds(wid * (_BPW // 8), _BPW // 8)
        for v, o in zip(vs, outs):
            pltpu.sync_copy(v, o.at[out_sl])

    return gather


_gather3 = _make_gather("three")
_gather1 = _make_gather("one")


_WB = 128
_LW = _WB * 1024


def _rne_bf16_bits(x):
    u = jax.lax.bitcast_convert_type(x, jnp.int32)
    return jax.lax.shift_right_logical(u + jnp.int32(0x8000), 16)


def _lin_body(in_ref, out_ref):
    ra = _rne_bf16_bits(in_ref[0, :_NP, :])
    rb = _rne_bf16_bits(in_ref[0, _NP:, :])
    w = jax.lax.bitcast_convert_type((rb << 16) | ra, jnp.float32)
    out_ref[...] = w.reshape(out_ref.shape)


def _linearize(embT3):
    return pl.pallas_call(
        _lin_body,
        grid=(pl.cdiv(_NB, _WB),),
        in_specs=[pl.BlockSpec((1, D, _LW), lambda b: (0, 0, b))],
        out_specs=pl.BlockSpec((_NP, _WB, 8, 128), lambda b: (0, b, 0, 0)),
        out_shape=jax.ShapeDtypeStruct((_NP, _NB, 8, 128), jnp.float32),
    )(embT3)


_BLK8 = 512


def _mlp_body(pred, umlp, imlp, w1a, w1b, b1, w2, b2, woa, wob, bo, out):
    h = jnp.dot(umlp[...], w1a[...], preferred_element_type=jnp.float32)
    h = h + jnp.dot(imlp[...], w1b[...], preferred_element_type=jnp.float32)
    h = jnp.maximum(h + b1[...], 0.0)
    h = jnp.dot(h, w2[...], preferred_element_type=jnp.float32) + b2[...]
    h = jnp.maximum(h, 0.0)
    r = jnp.dot(pred[...], woa[...], preferred_element_type=jnp.float32)
    r = r + jnp.dot(h, wob[...], preferred_element_type=jnp.float32)
    out[...] = r + bo[...]


def _mlp(pred, umlp, imlp, w1a, w1b, b1, w2, b2, woa, wob, bo):
    row = pl.BlockSpec((_BLK8, 128), lambda i: (i, 0))

    def full(a):
        return pl.BlockSpec(a.shape, lambda i: (0,) * a.ndim)

    return pl.pallas_call(
        _mlp_body,
        grid=(B // 8 // _BLK8,),
        in_specs=[row, row, row,
                  full(w1a), full(w1b), full(b1), full(w2), full(b2),
                  full(woa), full(wob), full(bo)],
        out_specs=pl.BlockSpec((_BLK8, 8), lambda i: (i, 0)),
        out_shape=jax.ShapeDtypeStruct((B // 8, 8), jnp.float32),
    )(pred, umlp, imlp, w1a, w1b, b1, w2, b2, woa, wob, bo)


def kernel(user_indices, item_indices, emb_user_mf, emb_item_mf,
           emb_user_mlp, emb_item_mlp, W1, b1, W2, b2, Wout, bout):
    uidx = user_indices.astype(jnp.int32)
    iidx = item_indices.astype(jnp.int32)
    def lin(emb):
        return _linearize(emb.T.reshape(1, D, NROWS)).reshape(_FROWS, D)

    fu_mf = lin(emb_user_mf)
    fi_mf = lin(emb_item_mf)
    fu_mlp = lin(emb_user_mlp)
    pred, umlp = _gather3(uidx, iidx, fu_mf, fi_mf, fu_mlp)
    fi_mlp = lin(emb_item_mlp)
    (imlp,) = _gather1(iidx, fi_mlp)
    eye8 = jnp.eye(8, dtype=jnp.float32)
    w1a = jnp.kron(eye8, W1[:, :D].T)
    w1b = jnp.kron(eye8, W1[:, D:].T)
    w2d = jnp.kron(eye8, W2.T)
    woa = jnp.kron(eye8, Wout[:, :D].T)
    wob = jnp.kron(eye8, Wout[:, D:].T)
    b1t = jnp.tile(b1, 8).reshape(1, -1)
    b2t = jnp.tile(b2, 8).reshape(1, -1)
    out8 = _mlp(pred, umlp, imlp, w1a, w1b, b1t, w2d, b2t, woa, wob,
                bout.reshape(1, 1))
    return out8.reshape(B, 1)


# speedup vs baseline: 1.0088x; 1.0088x over previous
import functools

import jax
import jax.numpy as jnp
from jax import lax
from jax.experimental import pallas as pl
from jax.experimental.pallas import tpu as pltpu
from jax.experimental.pallas import tpu_sc as plsc

B = 16384
D = 16
NROWS = 1000000
_NC = 2
_NS = 16
_NW = _NC * _NS
_BPW = B // _NW
_CHUNK = 64
_NCHUNK = _BPW // _CHUNK
_KG = _CHUNK // 16

_NP = D // 2
_NB = (NROWS + 1023) // 1024
_FPAD = _NB * 1024
_FROWS = _FPAD * _NP // 16
_JC = [j * (_FPAD // 16) for j in range(_NP)]

_mesh = plsc.VectorSubcoreMesh(core_axis_name="c", subcore_axis_name="s")


def _burst_base(iv):
    return iv >> 4


def _make_gather(mode):
    three = mode == "three"
    n_out = {"three": 2, "one": 1, "mf": 1, "mlp": 2}[mode]
    n_tab = {"three": 3, "one": 1, "mf": 2, "mlp": 2}[mode]
    n_idx = {"three": 2, "one": 1, "mf": 2, "mlp": 2}[mode]

    @functools.partial(
        pl.kernel,
        mesh=_mesh,
        compiler_params=pltpu.CompilerParams(
            use_tc_tiling_on_sc=False, needs_layout_passes=False),
        out_type=[jax.ShapeDtypeStruct((B // 8, 128), jnp.float32)] * n_out,
        scratch_types=(
            [pltpu.VMEM((_BPW,), jnp.int32)] * n_idx
            + [pltpu.VMEM((2, _CHUNK * _NP), jnp.int32)] * n_idx
            + [pltpu.VMEM((2, _CHUNK * _NP, 16), jnp.float32)] * n_tab
            + [pltpu.VMEM((_BPW // 8, 128), jnp.float32)] * n_out
            + [pltpu.SemaphoreType.DMA]
        ),
    )
    def gather(*args):
        pos = 0

        def take(n):
            nonlocal pos
            out = args[pos:pos + n]
            pos += n
            return out

        idx_hbm = take(n_idx)
        tabs = take(n_tab)
        outs = take(n_out)
        idx_v = take(n_idx)
        ibs = take(n_idx)
        gs = take(n_tab)
        vs = take(n_out)
        (sem,) = take(1)

        wid = lax.axis_index("s") * _NC + lax.axis_index("c")
        base = wid * _BPW
        for h, v in zip(idx_hbm, idx_v):
            pltpu.sync_copy(h.at[pl.ds(base, _BPW)], v)

        iota = lax.iota(jnp.int32, 16)
        tab_idx = {"three": (0, 1, 0), "one": (0,),
                   "mf": (0, 1), "mlp": (0, 1)}[mode]

        def unpack(w):
            wi = plsc.bitcast(w, jnp.int32)
            lo = plsc.bitcast(wi << 16, jnp.float32)
            hi = plsc.bitcast(wi & jnp.int32(-65536), jnp.float32)
            return lo, hi

        def build_and_fire(c):
            p = c % 2
            c0 = c * _CHUNK
            for k in range(_KG):
                for ib, iv_ref in zip(ibs, idx_v):
                    bv = _burst_base(iv_ref[pl.ds(c0 + k * 16, 16)])
                    for j in range(_NP):
                        ib[p, pl.ds(j * _CHUNK + k * 16, 16)] = bv + _JC[j]
            return [pltpu.async_copy(t.at[ibs[s].at[p]], g.at[p], sem)
                    for t, g, s in zip(tabs, gs, tab_idx)]

        def extract(c):
            p = c % 2
            c0 = c * _CHUNK
            for k in range(_KG):
                lanes = [v[pl.ds(c0 + k * 16, 16)] & 15 for v in idx_v]
                rows = c0 + k * 16 + iota
                r2 = rows >> 3
                cb = (rows & 7) << 4
                for j in range(_NP):
                    gr = j * _CHUNK + k * 16 + iota
                    pairs = [
                        unpack(plsc.load_gather(gs[t].at[p], [gr,
                                                             lanes[s]]))
                        for t, s in enumerate(tab_idx)]
                    if mode == "three":
                        (u0, u1), (i0, i1), (m0, m1) = pairs
                        plsc.store_scatter(vs[0], [r2, cb + j], u0 * i0)
                        plsc.store_scatter(vs[0], [r2, cb + (j + _NP)],
                                           u1 * i1)
                        plsc.store_scatter(vs[1], [r2, cb + j], m0)
                        plsc.store_scatter(vs[1], [r2, cb + (j + _NP)], m1)
                    elif mode == "mf":
                        (u0, u1), (i0, i1) = pairs
                        plsc.store_scatter(vs[0], [r2, cb + j], u0 * i0)
                        plsc.store_scatter(vs[0], [r2, cb + (j + _NP)],
                                           u1 * i1)
                    else:
                        for v, (x0, x1) in zip(vs, pairs):
                            plsc.store_scatter(v, [r2, cb + j], x0)
                            plsc.store_scatter(v, [r2, cb + (j + _NP)], x1)

        cps = build_and_fire(0)
        for c in range(1, _NCHUNK):
            nxt = build_and_fire(c)
            for cp in cps:
                cp.wait()
            extract(c - 1)
            cps = nxt
        for cp in cps:
            cp.wait()
        extract(_NCHUNK - 1)

        out_sl = pl.ds(wid * (_BPW // 8), _BPW // 8)
        for v, o in zip(vs, outs):
            pltpu.sync_copy(v, o.at[out_sl])

    return gather


_gather_mf = _make_gather("mf")
_gather_mlp = _make_gather("mlp")


_WB = 128
_LW = _WB * 1024


def _rne_bf16_bits(x):
    u = jax.lax.bitcast_convert_type(x, jnp.int32)
    return jax.lax.shift_right_logical(u + jnp.int32(0x8000), 16)


def _lin_body(in_ref, out_ref):
    ra = _rne_bf16_bits(in_ref[0, :_NP, :])
    rb = _rne_bf16_bits(in_ref[0, _NP:, :])
    w = jax.lax.bitcast_convert_type((rb << 16) | ra, jnp.float32)
    out_ref[...] = w.reshape(out_ref.shape)


def _linearize(embT3):
    return pl.pallas_call(
        _lin_body,
        grid=(pl.cdiv(_NB, _WB),),
        in_specs=[pl.BlockSpec((1, D, _LW), lambda b: (0, 0, b))],
        out_specs=pl.BlockSpec((_NP, _WB, 8, 128), lambda b: (0, b, 0, 0)),
        out_shape=jax.ShapeDtypeStruct((_NP, _NB, 8, 128), jnp.float32),
    )(embT3)


_BLK8 = 512


def _mlp_body(pred, umlp, imlp, w1a, w1b, b1, w2, b2, woa, wob, bo, out):
    h = jnp.dot(umlp[...], w1a[...], preferred_element_type=jnp.float32)
    h = h + jnp.dot(imlp[...], w1b[...], preferred_element_type=jnp.float32)
    h = jnp.maximum(h + b1[...], 0.0)
    h = jnp.dot(h, w2[...], preferred_element_type=jnp.float32) + b2[...]
    h = jnp.maximum(h, 0.0)
    r = jnp.dot(pred[...], woa[...], preferred_element_type=jnp.float32)
    r = r + jnp.dot(h, wob[...], preferred_element_type=jnp.float32)
    out[...] = r + bo[...]


def _mlp(pred, umlp, imlp, w1a, w1b, b1, w2, b2, woa, wob, bo):
    row = pl.BlockSpec((_BLK8, 128), lambda i: (i, 0))

    def full(a):
        return pl.BlockSpec(a.shape, lambda i: (0,) * a.ndim)

    return pl.pallas_call(
        _mlp_body,
        grid=(B // 8 // _BLK8,),
        in_specs=[row, row, row,
                  full(w1a), full(w1b), full(b1), full(w2), full(b2),
                  full(woa), full(wob), full(bo)],
        out_specs=pl.BlockSpec((_BLK8, 8), lambda i: (i, 0)),
        out_shape=jax.ShapeDtypeStruct((B // 8, 8), jnp.float32),
    )(pred, umlp, imlp, w1a, w1b, b1, w2, b2, woa, wob, bo)


def kernel(user_indices, item_indices, emb_user_mf, emb_item_mf,
           emb_user_mlp, emb_item_mlp, W1, b1, W2, b2, Wout, bout):
    uidx = user_indices.astype(jnp.int32)
    iidx = item_indices.astype(jnp.int32)
    def lin(emb):
        return _linearize(emb.T.reshape(1, D, NROWS)).reshape(_FROWS, D)

    fu_mf = lin(emb_user_mf)
    fi_mf = lin(emb_item_mf)
    (pred,) = _gather_mf(uidx, iidx, fu_mf, fi_mf)
    fu_mlp = lin(emb_user_mlp)
    fi_mlp = lin(emb_item_mlp)
    umlp, imlp = _gather_mlp(uidx, iidx, fu_mlp, fi_mlp)
    eye8 = jnp.eye(8, dtype=jnp.float32)
    w1a = jnp.kron(eye8, W1[:, :D].T)
    w1b = jnp.kron(eye8, W1[:, D:].T)
    w2d = jnp.kron(eye8, W2.T)
    woa = jnp.kron(eye8, Wout[:, :D].T)
    wob = jnp.kron(eye8, Wout[:, D:].T)
    b1t = jnp.tile(b1, 8).reshape(1, -1)
    b2t = jnp.tile(b2, 8).reshape(1, -1)
    out8 = _mlp(pred, umlp, imlp, w1a, w1b, b1t, w2d, b2t, woa, wob,
                bout.reshape(1, 1))
    return out8.reshape(B, 1)


# speedup vs baseline: 1.0180x; 1.0091x over previous
import functools

import jax
import jax.numpy as jnp
from jax import lax
from jax.experimental import pallas as pl
from jax.experimental.pallas import tpu as pltpu
from jax.experimental.pallas import tpu_sc as plsc

B = 16384
D = 16
NROWS = 1000000
_NC = 2
_NS = 16
_NW = _NC * _NS
_BPW = B // _NW
_CHUNK = 128
_NCHUNK = _BPW // _CHUNK
_KG = _CHUNK // 16

_NP = D // 2
_NB = (NROWS + 1023) // 1024
_FPAD = _NB * 1024
_FROWS = _FPAD * _NP // 16
_JC = [j * (_FPAD // 16) for j in range(_NP)]

_mesh = plsc.VectorSubcoreMesh(core_axis_name="c", subcore_axis_name="s")


def _burst_base(iv):
    return iv >> 4


def _make_gather(mode):
    three = mode == "three"
    n_out = {"three": 2, "one": 1, "mf": 1, "mlp": 2}[mode]
    n_tab = {"three": 3, "one": 1, "mf": 2, "mlp": 2}[mode]
    n_idx = {"three": 2, "one": 1, "mf": 2, "mlp": 2}[mode]

    @functools.partial(
        pl.kernel,
        mesh=_mesh,
        compiler_params=pltpu.CompilerParams(
            use_tc_tiling_on_sc=False, needs_layout_passes=False),
        out_type=[jax.ShapeDtypeStruct((B // 8, 128), jnp.float32)] * n_out,
        scratch_types=(
            [pltpu.VMEM((_BPW,), jnp.int32)] * n_idx
            + [pltpu.VMEM((2, _CHUNK * _NP), jnp.int32)] * n_idx
            + [pltpu.VMEM((2, _CHUNK * _NP, 16), jnp.float32)] * n_tab
            + [pltpu.VMEM((_BPW // 8, 128), jnp.float32)] * n_out
            + [pltpu.SemaphoreType.DMA]
        ),
    )
    def gather(*args):
        pos = 0

        def take(n):
            nonlocal pos
            out = args[pos:pos + n]
            pos += n
            return out

        idx_hbm = take(n_idx)
        tabs = take(n_tab)
        outs = take(n_out)
        idx_v = take(n_idx)
        ibs = take(n_idx)
        gs = take(n_tab)
        vs = take(n_out)
        (sem,) = take(1)

        wid = lax.axis_index("s") * _NC + lax.axis_index("c")
        base = wid * _BPW
        for h, v in zip(idx_hbm, idx_v):
            pltpu.sync_copy(h.at[pl.ds(base, _BPW)], v)

        iota = lax.iota(jnp.int32, 16)
        tab_idx = {"three": (0, 1, 0), "one": (0,),
                   "mf": (0, 1), "mlp": (0, 1)}[mode]

        def unpack(w):
            wi = plsc.bitcast(w, jnp.int32)
            lo = plsc.bitcast(wi << 16, jnp.float32)
            hi = plsc.bitcast(wi & jnp.int32(-65536), jnp.float32)
            return lo, hi

        def build_and_fire(c):
            p = c % 2
            c0 = c * _CHUNK
            for k in range(_KG):
                for ib, iv_ref in zip(ibs, idx_v):
                    bv = _burst_base(iv_ref[pl.ds(c0 + k * 16, 16)])
                    for j in range(_NP):
                        ib[p, pl.ds(j * _CHUNK + k * 16, 16)] = bv + _JC[j]
            return [pltpu.async_copy(t.at[ibs[s].at[p]], g.at[p], sem)
                    for t, g, s in zip(tabs, gs, tab_idx)]

        def extract(c):
            p = c % 2
            c0 = c * _CHUNK
            for k in range(_KG):
                lanes = [v[pl.ds(c0 + k * 16, 16)] & 15 for v in idx_v]
                rows = c0 + k * 16 + iota
                r2 = rows >> 3
                cb = (rows & 7) << 4
                for j in range(_NP):
                    gr = j * _CHUNK + k * 16 + iota
                    pairs = [
                        unpack(plsc.load_gather(gs[t].at[p], [gr,
                                                             lanes[s]]))
                        for t, s in enumerate(tab_idx)]
                    if mode == "three":
                        (u0, u1), (i0, i1), (m0, m1) = pairs
                        plsc.store_scatter(vs[0], [r2, cb + j], u0 * i0)
                        plsc.store_scatter(vs[0], [r2, cb + (j + _NP)],
                                           u1 * i1)
                        plsc.store_scatter(vs[1], [r2, cb + j], m0)
                        plsc.store_scatter(vs[1], [r2, cb + (j + _NP)], m1)
                    elif mode == "mf":
                        (u0, u1), (i0, i1) = pairs
                        plsc.store_scatter(vs[0], [r2, cb + j], u0 * i0)
                        plsc.store_scatter(vs[0], [r2, cb + (j + _NP)],
                                           u1 * i1)
                    else:
                        for v, (x0, x1) in zip(vs, pairs):
                            plsc.store_scatter(v, [r2, cb + j], x0)
                            plsc.store_scatter(v, [r2, cb + (j + _NP)], x1)

        cps = build_and_fire(0)
        for c in range(1, _NCHUNK):
            nxt = build_and_fire(c)
            for cp in cps:
                cp.wait()
            extract(c - 1)
            cps = nxt
        for cp in cps:
            cp.wait()
        extract(_NCHUNK - 1)

        out_sl = pl.ds(wid * (_BPW // 8), _BPW // 8)
        for v, o in zip(vs, outs):
            pltpu.sync_copy(v, o.at[out_sl])

    return gather


_gather_mf = _make_gather("mf")
_gather_mlp = _make_gather("mlp")


_WB = 128
_LW = _WB * 1024


def _rne_bf16_bits(x):
    u = jax.lax.bitcast_convert_type(x, jnp.int32)
    return jax.lax.shift_right_logical(u + jnp.int32(0x8000), 16)


def _lin_body(in_ref, out_ref):
    ra = _rne_bf16_bits(in_ref[0, :_NP, :])
    rb = _rne_bf16_bits(in_ref[0, _NP:, :])
    w = jax.lax.bitcast_convert_type((rb << 16) | ra, jnp.float32)
    out_ref[...] = w.reshape(out_ref.shape)


def _linearize(embT3):
    return pl.pallas_call(
        _lin_body,
        grid=(pl.cdiv(_NB, _WB),),
        in_specs=[pl.BlockSpec((1, D, _LW), lambda b: (0, 0, b))],
        out_specs=pl.BlockSpec((_NP, _WB, 8, 128), lambda b: (0, b, 0, 0)),
        out_shape=jax.ShapeDtypeStruct((_NP, _NB, 8, 128), jnp.float32),
    )(embT3)


_BLK8 = 512


def _mlp_body(pred, umlp, imlp, w1a, w1b, b1, w2, b2, woa, wob, bo, out):
    h = jnp.dot(umlp[...], w1a[...], preferred_element_type=jnp.float32)
    h = h + jnp.dot(imlp[...], w1b[...], preferred_element_type=jnp.float32)
    h = jnp.maximum(h + b1[...], 0.0)
    h = jnp.dot(h, w2[...], preferred_element_type=jnp.float32) + b2[...]
    h = jnp.maximum(h, 0.0)
    r = jnp.dot(pred[...], woa[...], preferred_element_type=jnp.float32)
    r = r + jnp.dot(h, wob[...], preferred_element_type=jnp.float32)
    out[...] = r + bo[...]


def _mlp(pred, umlp, imlp, w1a, w1b, b1, w2, b2, woa, wob, bo):
    row = pl.BlockSpec((_BLK8, 128), lambda i: (i, 0))

    def full(a):
        return pl.BlockSpec(a.shape, lambda i: (0,) * a.ndim)

    return pl.pallas_call(
        _mlp_body,
        grid=(B // 8 // _BLK8,),
        in_specs=[row, row, row,
                  full(w1a), full(w1b), full(b1), full(w2), full(b2),
                  full(woa), full(wob), full(bo)],
        out_specs=pl.BlockSpec((_BLK8, 8), lambda i: (i, 0)),
        out_shape=jax.ShapeDtypeStruct((B // 8, 8), jnp.float32),
    )(pred, umlp, imlp, w1a, w1b, b1, w2, b2, woa, wob, bo)


def kernel(user_indices, item_indices, emb_user_mf, emb_item_mf,
           emb_user_mlp, emb_item_mlp, W1, b1, W2, b2, Wout, bout):
    uidx = user_indices.astype(jnp.int32)
    iidx = item_indices.astype(jnp.int32)
    def lin(emb):
        return _linearize(emb.T.reshape(1, D, NROWS)).reshape(_FROWS, D)

    fu_mf = lin(emb_user_mf)
    fi_mf = lin(emb_item_mf)
    (pred,) = _gather_mf(uidx, iidx, fu_mf, fi_mf)
    fu_mlp = lin(emb_user_mlp)
    fi_mlp = lin(emb_item_mlp)
    umlp, imlp = _gather_mlp(uidx, iidx, fu_mlp, fi_mlp)
    eye8 = jnp.eye(8, dtype=jnp.float32)
    w1a = jnp.kron(eye8, W1[:, :D].T)
    w1b = jnp.kron(eye8, W1[:, D:].T)
    w2d = jnp.kron(eye8, W2.T)
    woa = jnp.kron(eye8, Wout[:, :D].T)
    wob = jnp.kron(eye8, Wout[:, D:].T)
    b1t = jnp.tile(b1, 8).reshape(1, -1)
    b2t = jnp.tile(b2, 8).reshape(1, -1)
    out8 = _mlp(pred, umlp, imlp, w1a, w1b, b1t, w2d, b2t, woa, wob,
                bout.reshape(1, 1))
    return out8.reshape(B, 1)
